# baseline (speedup 1.0000x reference)
import functools

import jax
import jax.numpy as jnp
from jax import lax
from jax.experimental import pallas as pl
from jax.experimental.pallas import tpu as pltpu
from jax.experimental.pallas import tpu_sc as plsc

_NUM_TOKENS = 4096 * 200
_FEATURES = 64
_NW = 32
_PER_W = _NUM_TOKENS // _NW
_K = 128
_NCHUNK = _PER_W // _K


def _make_kernel():
    mesh = plsc.VectorSubcoreMesh(core_axis_name="c", subcore_axis_name="s")

    @functools.partial(
        pl.kernel,
        mesh=mesh,
        compiler_params=pltpu.CompilerParams(use_tc_tiling_on_sc=False),
        out_type=jax.ShapeDtypeStruct((_NUM_TOKENS, _FEATURES), jnp.float32),
        scratch_types=[
            pltpu.VMEM((1, _K), jnp.int32),
            pltpu.VMEM((_K, _FEATURES), jnp.float32),
            pltpu.SemaphoreType.DMA,
        ],
    )
    def emb_kernel(idx_hbm, table_hbm, out_hbm, idx_v, rows_v, sem):
        wid = lax.axis_index("s") * 2 + lax.axis_index("c")
        base = wid * _PER_W

        def body(i, carry):
            off = base + i * _K
            pltpu.sync_copy(idx_hbm.at[pl.ds(off, _K)], idx_v.at[0])
            pltpu.async_copy(table_hbm.at[idx_v.at[0]], rows_v, sem).wait()
            pltpu.sync_copy(rows_v, out_hbm.at[pl.ds(off, _K)])
            return carry

        lax.fori_loop(0, _NCHUNK, body, 0)

    return emb_kernel


_emb = _make_kernel()


def kernel(tokens, embedding):
    flat = tokens.reshape(_NUM_TOKENS)
    out = _emb(flat, embedding)
    return out.reshape(tokens.shape[0], tokens.shape[1], _FEATURES)


# speedup vs baseline: 1.1959x; 1.1959x over previous
import functools

import jax
import jax.numpy as jnp
from jax import lax
from jax.experimental import pallas as pl
from jax.experimental.pallas import tpu as pltpu
from jax.experimental.pallas import tpu_sc as plsc

_NUM_TOKENS = 4096 * 200
_FEATURES = 64
_NW = 32
_PER_W = _NUM_TOKENS // _NW
_K = 128
_KSUB = 4
_C = _K * _KSUB
_NCHUNK = _PER_W // _C
_NROWS = _PER_W // _K


def _make_kernel():
    mesh = plsc.VectorSubcoreMesh(core_axis_name="c", subcore_axis_name="s")

    @functools.partial(
        pl.kernel,
        mesh=mesh,
        compiler_params=pltpu.CompilerParams(use_tc_tiling_on_sc=False),
        out_type=jax.ShapeDtypeStruct((_NUM_TOKENS, _FEATURES), jnp.float32),
        scratch_types=[
            pltpu.VMEM((_NROWS, _K), jnp.int32),
            pltpu.VMEM((_C, _FEATURES), jnp.float32),
            pltpu.VMEM((_C, _FEATURES), jnp.float32),
            pltpu.SemaphoreType.DMA,
            pltpu.SemaphoreType.DMA,
        ],
    )
    def emb_kernel(idx_hbm, table_hbm, out_hbm, idx_v, buf0, buf1, sem0, sem1):
        wid = lax.axis_index("s") * 2 + lax.axis_index("c")
        base = wid * _PER_W

        pltpu.sync_copy(idx_hbm.at[wid], idx_v)

        def fire(c, buf, sem):
            for s in range(_KSUB):
                pltpu.async_copy(
                    table_hbm.at[idx_v.at[c * _KSUB + s]],
                    buf.at[pl.ds(s * _K, _K)],
                    sem,
                )

        def drain(buf, sem):
            for s in range(_KSUB):
                pltpu.make_async_copy(
                    table_hbm.at[idx_v.at[0]],
                    buf.at[pl.ds(s * _K, _K)],
                    sem,
                ).wait()

        def write(c, buf):
            pltpu.sync_copy(buf, out_hbm.at[pl.ds(base + c * _C, _C)])

        fire(0, buf0, sem0)
        nj = _NCHUNK // 2

        def body(j, carry):
            fire(2 * j + 1, buf1, sem1)
            drain(buf0, sem0)
            write(2 * j, buf0)

            @pl.when(j < nj - 1)
            def _():
                fire(2 * j + 2, buf0, sem0)

            drain(buf1, sem1)
            write(2 * j + 1, buf1)
            return carry

        lax.fori_loop(0, nj, body, 0)

    return emb_kernel


_emb = _make_kernel()


def kernel(tokens, embedding):
    idx = tokens.reshape(_NW, _NROWS, _K)
    out = _emb(idx, embedding)
    return out.reshape(tokens.shape[0], tokens.shape[1], _FEATURES)
